# baseline (speedup 1.0000x reference)
import functools

import jax
import jax.numpy as jnp
from jax import lax
from jax.experimental import pallas as pl
from jax.experimental.pallas import tpu as pltpu
from jax.experimental.pallas import tpu_sc as plsc

_N = 10000
_E = 320000
_D = 128
_B = 16

_NC = 2
_NS = 16
_NW = _NC * _NS

_NP = 10240
_RPT = _NP // _NS

_K = 128
_NCH = 80
_EP = _NW * _NCH * _K

_EROWS = _E // 128
_EPROWS = _EP // 128
_PBLK = 320

_ROWS = 1280
_GRID = _NP // _ROWS

_mesh = plsc.VectorSubcoreMesh(core_axis_name="c", subcore_axis_name="s")


def _prep_body(src_ref, dst_ref, pk_ref, d_ref):
    i = pl.program_id(0)
    rid = (lax.broadcasted_iota(jnp.int32, (_PBLK, 128), 0) * 128
           + lax.broadcasted_iota(jnp.int32, (_PBLK, 128), 1)
           + i * _PBLK * 128)
    valid = rid < _E
    pad_i = _N + rid % (_NP - _N)
    s = jnp.where(valid, src_ref[0], pad_i)
    d = jnp.where(valid, dst_ref[0], pad_i)
    pk_ref[...] = s | (d << 16)
    d_ref[...] = d


def _prep_call(edge3):
    return pl.pallas_call(
        _prep_body,
        grid=(_EPROWS // _PBLK,),
        in_specs=[pl.BlockSpec((1, _PBLK, 128), lambda i: (0, i, 0)),
                  pl.BlockSpec((1, _PBLK, 128), lambda i: (1, i, 0))],
        out_specs=[pl.BlockSpec((_PBLK, 128), lambda i: (i, 0)),
                   pl.BlockSpec((_PBLK, 128), lambda i: (i, 0))],
        out_shape=[jax.ShapeDtypeStruct((_EPROWS, 128), jnp.int32),
                   jax.ShapeDtypeStruct((_EPROWS, 128), jnp.int32)],
    )(edge3, edge3)


def _deg_body(dst_hbm, out_hbm, dst_v, ones_v, zero_v, acc_sh, sem):
    cid = lax.axis_index("c")
    sid = lax.axis_index("s")
    w = cid * _NS + sid
    pltpu.sync_copy(dst_hbm.at[w], dst_v)
    for i in range(_K // 16):
        ones_v[pl.ds(i * 16, 16)] = jnp.ones((16,), jnp.float32)
    for i in range(_RPT // 16):
        zero_v[pl.ds(i * 16, 16)] = jnp.zeros((16,), jnp.float32)
    pltpu.sync_copy(zero_v, acc_sh.at[pl.ds(sid * _RPT, _RPT)])
    plsc.subcore_barrier()

    def fire(j, carry):
        pltpu.async_copy(ones_v, acc_sh.at[dst_v.at[j]], sem, add=True)
        return carry

    lax.fori_loop(0, _NCH, fire, 0)

    def drain(j, carry):
        pltpu.make_async_copy(ones_v, acc_sh.at[dst_v.at[0]], sem).wait()
        return carry

    lax.fori_loop(0, _NCH, drain, 0)
    plsc.subcore_barrier()
    pltpu.sync_copy(acc_sh.at[pl.ds(sid * _RPT, _RPT)],
                    out_hbm.at[cid, pl.ds(sid * _RPT, _RPT)])


_deg_call = functools.partial(
    pl.kernel,
    out_type=jax.ShapeDtypeStruct((_NC, _NP), jnp.float32),
    mesh=_mesh,
    scratch_types=[
        pltpu.VMEM((_NCH, _K), jnp.int32),
        pltpu.VMEM((_K,), jnp.float32),
        pltpu.VMEM((_RPT,), jnp.float32),
        pltpu.VMEM_SHARED((_NP,), jnp.float32),
        pltpu.SemaphoreType.DMA,
    ],
)(_deg_body)


def _unpack16(packed_v, j, out_ref, hi):
    for i in range(_K // 16):
        wv = packed_v[j, pl.ds(i * 16, 16)]
        if hi:
            v = lax.shift_right_logical(wv, 16)
        else:
            v = lax.bitwise_and(wv, 0xFFFF)
        out_ref[pl.ds(i * 16, 16)] = v


def _spmm_body(xw_hbm, packed_hbm, out_hbm, packed_v, src_a, src_b, dst_c,
               buf_a, buf_b, acc_sh, sem_a, sem_b):
    cid = lax.axis_index("c")
    sid = lax.axis_index("s")
    w = cid * _NS + sid
    pltpu.sync_copy(packed_hbm.at[w], packed_v)
    pltpu.sync_copy(xw_hbm.at[pl.ds(sid * _RPT, _RPT)],
                    acc_sh.at[pl.ds(sid * _RPT, _RPT)])
    plsc.subcore_barrier()

    bufs = (buf_a, buf_b)
    srcs = (src_a, src_b)
    sems = (sem_a, sem_b)

    def gstart(j, b):
        _unpack16(packed_v, j, srcs[b], hi=False)
        pltpu.async_copy(xw_hbm.at[srcs[b]], bufs[b], sems[b])

    def gwait(b):
        pltpu.make_async_copy(xw_hbm.at[srcs[b]], bufs[b], sems[b]).wait()

    gstart(0, 0)
    gstart(1, 1)

    def outer(g, carry):
        j0 = 2 * g
        for b in range(2):
            j = j0 + b
            gwait(b)
            _unpack16(packed_v, j, dst_c, hi=True)
            pltpu.sync_copy(bufs[b], acc_sh.at[dst_c], add=True)
            nj = j + 2

            @pl.when(nj < _NCH)
            def _():
                gstart(nj, b)
        return carry

    lax.fori_loop(0, _NCH // 2, outer, 0)
    plsc.subcore_barrier()
    pltpu.sync_copy(acc_sh.at[pl.ds(sid * _RPT, _RPT)],
                    out_hbm.at[cid, pl.ds(sid * _RPT, _RPT)])


def _make_spmm():
    return functools.partial(
        pl.kernel,
        out_type=jax.ShapeDtypeStruct((_NC, _NP, _D), jnp.float32),
        mesh=_mesh,
        scratch_types=[
            pltpu.VMEM((_NCH, _K), jnp.int32),
            pltpu.VMEM((_K,), jnp.int32),
            pltpu.VMEM((_K,), jnp.int32),
            pltpu.VMEM((_K,), jnp.int32),
            pltpu.VMEM((_K, _D), jnp.float32),
            pltpu.VMEM((_K, _D), jnp.float32),
            pltpu.VMEM_SHARED((_NP, _D), jnp.float32),
            pltpu.SemaphoreType.DMA,
            pltpu.SemaphoreType.DMA,
        ],
    )(_spmm_body)


def _xw_body(x_ref, w_ref, p0_ref, p1_ref, o_ref):
    dinv = lax.rsqrt(1.0 + p0_ref[...] + p1_ref[...])
    o_ref[...] = jnp.dot(x_ref[...], w_ref[...],
                         preferred_element_type=jnp.float32) * dinv


def _mid_body(s_ref, xwp_ref, p0_ref, p1_ref, b_ref, w_ref, o_ref):
    dinv = lax.rsqrt(1.0 + p0_ref[...] + p1_ref[...])
    z = (s_ref[0] + s_ref[1] - xwp_ref[...]) * dinv + b_ref[...]
    h = jnp.where(z > 0, z, jnp.exp(z) - 1.0)
    o_ref[...] = jnp.dot(h, w_ref[...],
                         preferred_element_type=jnp.float32) * dinv


def _pool_body(s_ref, xwp_ref, p0_ref, p1_ref, b_ref, bid_ref,
               g_ref, acc, cnt):
    i = pl.program_id(0)

    @pl.when(i == 0)
    def _init():
        acc[...] = jnp.zeros_like(acc)
        cnt[...] = jnp.zeros_like(cnt)

    dinv = lax.rsqrt(1.0 + p0_ref[...] + p1_ref[...])
    z = (s_ref[0] + s_ref[1] - xwp_ref[...]) * dinv + b_ref[...]
    h = jnp.where(z > 0, z, jnp.exp(z) - 1.0)
    rid = lax.broadcasted_iota(jnp.int32, (_ROWS, 1), 0) + i * _ROWS
    live = (rid < _N).astype(jnp.float32)
    h = h * live
    onehot = (bid_ref[...] ==
              lax.broadcasted_iota(jnp.int32, (_ROWS, _B), 1)).astype(
                  jnp.float32)
    acc[...] += lax.dot_general(onehot, h, (((0,), (0,)), ((), ())),
                                preferred_element_type=jnp.float32)
    cnt[...] += lax.dot_general(onehot, live, (((0,), (0,)), ((), ())),
                                preferred_element_type=jnp.float32)

    @pl.when(i == _GRID - 1)
    def _fin():
        g_ref[...] = acc[...] / jnp.maximum(cnt[...], 1.0)


def _row_spec(cols):
    return pl.BlockSpec((_ROWS, cols), lambda i: (i, 0))


def _const_spec(shape):
    return pl.BlockSpec(shape, lambda i: (0, 0))


_s_spec = pl.BlockSpec((_NC, _ROWS, _D), lambda i: (0, i, 0))


def kernel(x, edge_index, batch, W1, b1, W2, b2):
    edge3 = edge_index.reshape(2, _EROWS, 128)
    packed2, dst2 = _prep_call(edge3)
    packed = packed2.reshape(_NW, _NCH, _K)
    dst3 = dst2.reshape(_NW, _NCH, _K)
    bid = batch.reshape(_N, 1)
    b1r = b1.reshape(1, _D)
    b2r = b2.reshape(1, _D)

    degp = _deg_call(dst3)
    p0 = degp[0].reshape(_NP, 1)
    p1 = degp[1].reshape(_NP, 1)

    xw1p = pl.pallas_call(
        _xw_body,
        grid=(_GRID,),
        in_specs=[_row_spec(_D), _const_spec((_D, _D)),
                  _row_spec(1), _row_spec(1)],
        out_specs=_row_spec(_D),
        out_shape=jax.ShapeDtypeStruct((_NP, _D), jnp.float32),
    )(x, W1, p0, p1)

    s = _make_spmm()(xw1p, packed)

    xw2p = pl.pallas_call(
        _mid_body,
        grid=(_GRID,),
        in_specs=[_s_spec, _row_spec(_D),
                  _row_spec(1), _row_spec(1),
                  _const_spec((1, _D)), _const_spec((_D, _D))],
        out_specs=_row_spec(_D),
        out_shape=jax.ShapeDtypeStruct((_NP, _D), jnp.float32),
    )(s, xw1p, p0, p1, b1r, W2)

    t = _make_spmm()(xw2p, packed)

    g = pl.pallas_call(
        _pool_body,
        grid=(_GRID,),
        in_specs=[_s_spec, _row_spec(_D),
                  _row_spec(1), _row_spec(1),
                  _const_spec((1, _D)), _row_spec(1)],
        out_specs=_const_spec((_B, _D)),
        out_shape=jax.ShapeDtypeStruct((_B, _D), jnp.float32),
        scratch_shapes=[pltpu.VMEM((_B, _D), jnp.float32),
                        pltpu.VMEM((_B, 1), jnp.float32)],
    )(t, xw2p, p0, p1, b2r, bid)

    return g


# speedup vs baseline: 1.0372x; 1.0372x over previous
import functools

import jax
import jax.numpy as jnp
from jax import lax
from jax.experimental import pallas as pl
from jax.experimental.pallas import tpu as pltpu
from jax.experimental.pallas import tpu_sc as plsc

_N = 10000
_E = 320000
_D = 128
_B = 16

_NC = 2
_NS = 16
_NW = _NC * _NS

_NP = 10240
_RPT = _NP // _NS

_K = 128
_NCH = 80
_EP = _NW * _NCH * _K

_EROWS = _E // 128
_EPROWS = _EP // 128
_PBLK = 320

_ROWS = 1280
_GRID = _NP // _ROWS

_mesh = plsc.VectorSubcoreMesh(core_axis_name="c", subcore_axis_name="s")


def _prep_body(src_ref, dst_ref, pk_ref, d_ref):
    i = pl.program_id(0)
    rid = (lax.broadcasted_iota(jnp.int32, (_PBLK, 128), 0) * 128
           + lax.broadcasted_iota(jnp.int32, (_PBLK, 128), 1)
           + i * _PBLK * 128)
    valid = rid < _E
    pad_i = _N + rid % (_NP - _N)
    s = jnp.where(valid, src_ref[0], pad_i)
    d = jnp.where(valid, dst_ref[0], pad_i)
    pk_ref[...] = s | (d << 16)
    d_ref[...] = d


def _prep_call(edge3):
    return pl.pallas_call(
        _prep_body,
        grid=(_EPROWS // _PBLK,),
        in_specs=[pl.BlockSpec((1, _PBLK, 128), lambda i: (0, i, 0)),
                  pl.BlockSpec((1, _PBLK, 128), lambda i: (1, i, 0))],
        out_specs=[pl.BlockSpec((_PBLK, 128), lambda i: (i, 0)),
                   pl.BlockSpec((_PBLK, 128), lambda i: (i, 0))],
        out_shape=[jax.ShapeDtypeStruct((_EPROWS, 128), jnp.int32),
                   jax.ShapeDtypeStruct((_EPROWS, 128), jnp.int32)],
    )(edge3, edge3)


def _deg_body(dst_hbm, out_hbm, dst_v, ones_v, zero_v, acc_sh, sem):
    cid = lax.axis_index("c")
    sid = lax.axis_index("s")
    w = cid * _NS + sid
    pltpu.sync_copy(dst_hbm.at[w], dst_v)
    for i in range(_K // 16):
        ones_v[pl.ds(i * 16, 16)] = jnp.ones((16,), jnp.float32)
    for i in range(_RPT // 16):
        zero_v[pl.ds(i * 16, 16)] = jnp.zeros((16,), jnp.float32)
    pltpu.sync_copy(zero_v, acc_sh.at[pl.ds(sid * _RPT, _RPT)])
    plsc.subcore_barrier()

    def fire(j, carry):
        pltpu.async_copy(ones_v, acc_sh.at[dst_v.at[j]], sem, add=True)
        return carry

    lax.fori_loop(0, _NCH, fire, 0)

    def drain(j, carry):
        pltpu.make_async_copy(ones_v, acc_sh.at[dst_v.at[0]], sem).wait()
        return carry

    lax.fori_loop(0, _NCH, drain, 0)
    plsc.subcore_barrier()
    pltpu.sync_copy(acc_sh.at[pl.ds(sid * _RPT, _RPT)],
                    out_hbm.at[cid, pl.ds(sid * _RPT, _RPT)])


_deg_call = functools.partial(
    pl.kernel,
    out_type=jax.ShapeDtypeStruct((_NC, _NP), jnp.float32),
    mesh=_mesh,
    scratch_types=[
        pltpu.VMEM((_NCH, _K), jnp.int32),
        pltpu.VMEM((_K,), jnp.float32),
        pltpu.VMEM((_RPT,), jnp.float32),
        pltpu.VMEM_SHARED((_NP,), jnp.float32),
        pltpu.SemaphoreType.DMA,
    ],
)(_deg_body)


def _unpack16(packed_v, j, out_ref, hi):
    for i in range(_K // 16):
        wv = packed_v[j, pl.ds(i * 16, 16)]
        if hi:
            v = lax.shift_right_logical(wv, 16)
        else:
            v = lax.bitwise_and(wv, 0xFFFF)
        out_ref[pl.ds(i * 16, 16)] = v


def _spmm_body(xw_hbm, packed_hbm, out_hbm, packed_v, src_a, src_b, dst_c,
               buf_a, buf_b, acc_sh, sem_a, sem_b):
    cid = lax.axis_index("c")
    sid = lax.axis_index("s")
    w = cid * _NS + sid
    pltpu.sync_copy(packed_hbm.at[w], packed_v)
    pltpu.sync_copy(xw_hbm.at[pl.ds(sid * _RPT, _RPT)],
                    acc_sh.at[pl.ds(sid * _RPT, _RPT)])
    plsc.subcore_barrier()

    bufs = (buf_a, buf_b)
    srcs = (src_a, src_b)
    sems = (sem_a, sem_b)

    def gstart(j, b):
        _unpack16(packed_v, j, srcs[b], hi=False)
        pltpu.async_copy(xw_hbm.at[srcs[b]], bufs[b], sems[b])

    def gwait(b):
        pltpu.make_async_copy(xw_hbm.at[srcs[b]], bufs[b], sems[b]).wait()

    gstart(0, 0)
    gstart(1, 1)

    def outer(g, carry):
        j0 = 2 * g
        for b in range(2):
            j = j0 + b
            gwait(b)
            _unpack16(packed_v, j, dst_c, hi=True)
            pltpu.sync_copy(bufs[b], acc_sh.at[dst_c], add=True)
            nj = j + 2

            @pl.when(nj < _NCH)
            def _():
                gstart(nj, b)
        return carry

    lax.fori_loop(0, _NCH // 2, outer, 0)
    plsc.subcore_barrier()
    pltpu.sync_copy(acc_sh.at[pl.ds(sid * _RPT, _RPT)],
                    out_hbm.at[cid, pl.ds(sid * _RPT, _RPT)])


def _make_spmm():
    return functools.partial(
        pl.kernel,
        out_type=jax.ShapeDtypeStruct((_NC, _NP, _D), jnp.float32),
        mesh=_mesh,
        scratch_types=[
            pltpu.VMEM((_NCH, _K), jnp.int32),
            pltpu.VMEM((_K,), jnp.int32),
            pltpu.VMEM((_K,), jnp.int32),
            pltpu.VMEM((_K,), jnp.int32),
            pltpu.VMEM((_K, _D), jnp.float32),
            pltpu.VMEM((_K, _D), jnp.float32),
            pltpu.VMEM_SHARED((_NP, _D), jnp.float32),
            pltpu.SemaphoreType.DMA,
            pltpu.SemaphoreType.DMA,
        ],
    )(_spmm_body)


def _dinv_col(p_ref):
    d = lax.rsqrt(1.0 + p_ref[0:1, :] + p_ref[1:2, :])
    return jnp.transpose(d, (1, 0))


def _xw_body(x_ref, w_ref, p_ref, o_ref):
    dinv = _dinv_col(p_ref)
    o_ref[...] = jnp.dot(x_ref[...], w_ref[...],
                         preferred_element_type=jnp.float32) * dinv


def _mid_body(s_ref, xwp_ref, p_ref, b_ref, w_ref, o_ref):
    dinv = _dinv_col(p_ref)
    z = (s_ref[0] + s_ref[1] - xwp_ref[...]) * dinv + b_ref[...]
    h = jnp.where(z > 0, z, jnp.exp(z) - 1.0)
    o_ref[...] = jnp.dot(h, w_ref[...],
                         preferred_element_type=jnp.float32) * dinv


def _pool_body(s_ref, xwp_ref, p_ref, b_ref, bid_ref,
               g_ref, acc, cnt):
    i = pl.program_id(0)

    @pl.when(i == 0)
    def _init():
        acc[...] = jnp.zeros_like(acc)
        cnt[...] = jnp.zeros_like(cnt)

    dinv = _dinv_col(p_ref)
    z = (s_ref[0] + s_ref[1] - xwp_ref[...]) * dinv + b_ref[...]
    h = jnp.where(z > 0, z, jnp.exp(z) - 1.0)
    rid = lax.broadcasted_iota(jnp.int32, (_ROWS, 1), 0) + i * _ROWS
    live = (rid < _N).astype(jnp.float32)
    h = h * live
    onehot = (bid_ref[...] ==
              lax.broadcasted_iota(jnp.int32, (_ROWS, _B), 1)).astype(
                  jnp.float32)
    acc[...] += lax.dot_general(onehot, h, (((0,), (0,)), ((), ())),
                                preferred_element_type=jnp.float32)
    cnt[...] += lax.dot_general(onehot, live, (((0,), (0,)), ((), ())),
                                preferred_element_type=jnp.float32)

    @pl.when(i == _GRID - 1)
    def _fin():
        g_ref[...] = acc[...] / jnp.maximum(cnt[...], 1.0)


def _row_spec(cols):
    return pl.BlockSpec((_ROWS, cols), lambda i: (i, 0))


def _const_spec(shape):
    return pl.BlockSpec(shape, lambda i: (0, 0))


_s_spec = pl.BlockSpec((_NC, _ROWS, _D), lambda i: (0, i, 0))


def kernel(x, edge_index, batch, W1, b1, W2, b2):
    edge3 = edge_index.reshape(2, _EROWS, 128)
    packed2, dst2 = _prep_call(edge3)
    packed = packed2.reshape(_NW, _NCH, _K)
    dst3 = dst2.reshape(_NW, _NCH, _K)
    bid = batch.reshape(_N, 1)
    b1r = b1.reshape(1, _D)
    b2r = b2.reshape(1, _D)

    degp = _deg_call(dst3)
    p_spec = pl.BlockSpec((_NC, _ROWS), lambda i: (0, i))

    xw1p = pl.pallas_call(
        _xw_body,
        grid=(_GRID,),
        in_specs=[_row_spec(_D), _const_spec((_D, _D)), p_spec],
        out_specs=_row_spec(_D),
        out_shape=jax.ShapeDtypeStruct((_NP, _D), jnp.float32),
    )(x, W1, degp)

    s = _make_spmm()(xw1p, packed)

    xw2p = pl.pallas_call(
        _mid_body,
        grid=(_GRID,),
        in_specs=[_s_spec, _row_spec(_D), p_spec,
                  _const_spec((1, _D)), _const_spec((_D, _D))],
        out_specs=_row_spec(_D),
        out_shape=jax.ShapeDtypeStruct((_NP, _D), jnp.float32),
    )(s, xw1p, degp, b1r, W2)

    t = _make_spmm()(xw2p, packed)

    g = pl.pallas_call(
        _pool_body,
        grid=(_GRID,),
        in_specs=[_s_spec, _row_spec(_D), p_spec,
                  _const_spec((1, _D)), _row_spec(1)],
        out_specs=_const_spec((_B, _D)),
        out_shape=jax.ShapeDtypeStruct((_B, _D), jnp.float32),
        scratch_shapes=[pltpu.VMEM((_B, _D), jnp.float32),
                        pltpu.VMEM((_B, 1), jnp.float32)],
    )(t, xw2p, degp, b2r, bid)

    return g


# speedup vs baseline: 1.0663x; 1.0280x over previous
import functools

import jax
import jax.numpy as jnp
from jax import lax
from jax.experimental import pallas as pl
from jax.experimental.pallas import tpu as pltpu
from jax.experimental.pallas import tpu_sc as plsc

_N = 10000
_E = 320000
_D = 128
_B = 16

_NC = 2
_NS = 16
_NW = _NC * _NS

_NP = 10240
_RPT = _NP // _NS

_K = 64
_NCH = 162
_EP = _NW * _NCH * _K

_EROWS = _E // 128
_EPROWS = _EP // 128
_PBLK = 288

_ROWS = 1280
_GRID = _NP // _ROWS

_mesh = plsc.VectorSubcoreMesh(core_axis_name="c", subcore_axis_name="s")


def _prep_body(src_ref, dst_ref, pk_ref, d_ref):
    i = pl.program_id(0)
    rid = (lax.broadcasted_iota(jnp.int32, (_PBLK, 128), 0) * 128
           + lax.broadcasted_iota(jnp.int32, (_PBLK, 128), 1)
           + i * _PBLK * 128)
    valid = rid < _E
    pad_i = _N + rid % (_NP - _N)
    s = jnp.where(valid, src_ref[0], pad_i)
    d = jnp.where(valid, dst_ref[0], pad_i)
    pk_ref[...] = s | (d << 16)
    d_ref[...] = d


def _prep_call(edge3):
    return pl.pallas_call(
        _prep_body,
        grid=(_EPROWS // _PBLK,),
        in_specs=[pl.BlockSpec((1, _PBLK, 128), lambda i: (0, i, 0)),
                  pl.BlockSpec((1, _PBLK, 128), lambda i: (1, i, 0))],
        out_specs=[pl.BlockSpec((_PBLK, 128), lambda i: (i, 0)),
                   pl.BlockSpec((_PBLK, 128), lambda i: (i, 0))],
        out_shape=[jax.ShapeDtypeStruct((_EPROWS, 128), jnp.int32),
                   jax.ShapeDtypeStruct((_EPROWS, 128), jnp.int32)],
    )(edge3, edge3)


def _deg_body(dst_hbm, out_hbm, dst_v, ones_v, zero_v, acc_sh, sem):
    cid = lax.axis_index("c")
    sid = lax.axis_index("s")
    w = cid * _NS + sid
    pltpu.sync_copy(dst_hbm.at[w], dst_v)
    for i in range(_K // 16):
        ones_v[pl.ds(i * 16, 16)] = jnp.ones((16,), jnp.float32)
    for i in range(_RPT // 16):
        zero_v[pl.ds(i * 16, 16)] = jnp.zeros((16,), jnp.float32)
    pltpu.sync_copy(zero_v, acc_sh.at[pl.ds(sid * _RPT, _RPT)])
    plsc.subcore_barrier()

    def fire(j, carry):
        pltpu.async_copy(ones_v, acc_sh.at[dst_v.at[j]], sem, add=True)
        return carry

    lax.fori_loop(0, _NCH, fire, 0)

    def drain(j, carry):
        pltpu.make_async_copy(ones_v, acc_sh.at[dst_v.at[0]], sem).wait()
        return carry

    lax.fori_loop(0, _NCH, drain, 0)
    plsc.subcore_barrier()
    pltpu.sync_copy(acc_sh.at[pl.ds(sid * _RPT, _RPT)],
                    out_hbm.at[cid, pl.ds(sid * _RPT, _RPT)])


_deg_call = functools.partial(
    pl.kernel,
    out_type=jax.ShapeDtypeStruct((_NC, _NP), jnp.float32),
    mesh=_mesh,
    scratch_types=[
        pltpu.VMEM((_NCH, _K), jnp.int32),
        pltpu.VMEM((_K,), jnp.float32),
        pltpu.VMEM((_RPT,), jnp.float32),
        pltpu.VMEM_SHARED((_NP,), jnp.float32),
        pltpu.SemaphoreType.DMA,
    ],
)(_deg_body)


def _unpack16(packed_v, j, out_ref, hi):
    for i in range(_K // 16):
        wv = packed_v[j, pl.ds(i * 16, 16)]
        if hi:
            v = lax.shift_right_logical(wv, 16)
        else:
            v = lax.bitwise_and(wv, 0xFFFF)
        out_ref[pl.ds(i * 16, 16)] = v


def _spmm_body(xw_hbm, packed_hbm, out_hbm, packed_v, src_a, src_b, src_c,
               dst_c, buf_a, buf_b, buf_c, acc_sh, sem_a, sem_b, sem_c):
    cid = lax.axis_index("c")
    sid = lax.axis_index("s")
    w = cid * _NS + sid
    pltpu.sync_copy(packed_hbm.at[w], packed_v)

    bufs = (buf_a, buf_b, buf_c)
    srcs = (src_a, src_b, src_c)
    sems = (sem_a, sem_b, sem_c)

    def gstart(j, b):
        _unpack16(packed_v, j, srcs[b], hi=False)
        pltpu.async_copy(xw_hbm.at[srcs[b]], bufs[b], sems[b])

    def gwait(b):
        pltpu.make_async_copy(xw_hbm.at[srcs[b]], bufs[b], sems[b]).wait()

    gstart(0, 0)
    gstart(1, 1)
    gstart(2, 2)
    pltpu.sync_copy(xw_hbm.at[pl.ds(sid * _RPT, _RPT)],
                    acc_sh.at[pl.ds(sid * _RPT, _RPT)])
    plsc.subcore_barrier()

    def outer(g, carry):
        j0 = 3 * g
        for b in range(3):
            j = j0 + b
            gwait(b)
            _unpack16(packed_v, j, dst_c, hi=True)
            pltpu.sync_copy(bufs[b], acc_sh.at[dst_c], add=True)
            nj = j + 3

            @pl.when(nj < _NCH)
            def _():
                gstart(nj, b)
        return carry

    lax.fori_loop(0, _NCH // 3, outer, 0)
    plsc.subcore_barrier()
    pltpu.sync_copy(acc_sh.at[pl.ds(sid * _RPT, _RPT)],
                    out_hbm.at[cid, pl.ds(sid * _RPT, _RPT)])


def _make_spmm():
    return functools.partial(
        pl.kernel,
        out_type=jax.ShapeDtypeStruct((_NC, _NP, _D), jnp.float32),
        mesh=_mesh,
        scratch_types=[
            pltpu.VMEM((_NCH, _K), jnp.int32),
            pltpu.VMEM((_K,), jnp.int32),
            pltpu.VMEM((_K,), jnp.int32),
            pltpu.VMEM((_K,), jnp.int32),
            pltpu.VMEM((_K,), jnp.int32),
            pltpu.VMEM((_K, _D), jnp.float32),
            pltpu.VMEM((_K, _D), jnp.float32),
            pltpu.VMEM((_K, _D), jnp.float32),
            pltpu.VMEM_SHARED((_NP, _D), jnp.float32),
            pltpu.SemaphoreType.DMA,
            pltpu.SemaphoreType.DMA,
            pltpu.SemaphoreType.DMA,
        ],
    )(_spmm_body)


def _dinv_col(p_ref):
    d = lax.rsqrt(1.0 + p_ref[0:1, :] + p_ref[1:2, :])
    return jnp.transpose(d, (1, 0))


def _xw_body(x_ref, w_ref, p_ref, o_ref):
    dinv = _dinv_col(p_ref)
    o_ref[...] = jnp.dot(x_ref[...], w_ref[...],
                         preferred_element_type=jnp.float32) * dinv


def _mid_body(s_ref, xwp_ref, p_ref, b_ref, w_ref, o_ref):
    dinv = _dinv_col(p_ref)
    z = (s_ref[0] + s_ref[1] - xwp_ref[...]) * dinv + b_ref[...]
    h = jnp.where(z > 0, z, jnp.exp(z) - 1.0)
    o_ref[...] = jnp.dot(h, w_ref[...],
                         preferred_element_type=jnp.float32) * dinv


def _pool_body(s_ref, xwp_ref, p_ref, b_ref, bid_ref,
               g_ref, acc, cnt):
    i = pl.program_id(0)

    @pl.when(i == 0)
    def _init():
        acc[...] = jnp.zeros_like(acc)
        cnt[...] = jnp.zeros_like(cnt)

    dinv = _dinv_col(p_ref)
    z = (s_ref[0] + s_ref[1] - xwp_ref[...]) * dinv + b_ref[...]
    h = jnp.where(z > 0, z, jnp.exp(z) - 1.0)
    rid = lax.broadcasted_iota(jnp.int32, (_ROWS, 1), 0) + i * _ROWS
    live = (rid < _N).astype(jnp.float32)
    h = h * live
    onehot = (bid_ref[...] ==
              lax.broadcasted_iota(jnp.int32, (_ROWS, _B), 1)).astype(
                  jnp.float32)
    acc[...] += lax.dot_general(onehot, h, (((0,), (0,)), ((), ())),
                                preferred_element_type=jnp.float32)
    cnt[...] += lax.dot_general(onehot, live, (((0,), (0,)), ((), ())),
                                preferred_element_type=jnp.float32)

    @pl.when(i == _GRID - 1)
    def _fin():
        g_ref[...] = acc[...] / jnp.maximum(cnt[...], 1.0)


def _row_spec(cols):
    return pl.BlockSpec((_ROWS, cols), lambda i: (i, 0))


def _const_spec(shape):
    return pl.BlockSpec(shape, lambda i: (0, 0))


_s_spec = pl.BlockSpec((_NC, _ROWS, _D), lambda i: (0, i, 0))


def kernel(x, edge_index, batch, W1, b1, W2, b2):
    edge3 = edge_index.reshape(2, _EROWS, 128)
    packed2, dst2 = _prep_call(edge3)
    packed = packed2.reshape(_NW, _NCH, _K)
    dst3 = dst2.reshape(_NW, _NCH, _K)
    bid = batch.reshape(_N, 1)
    b1r = b1.reshape(1, _D)
    b2r = b2.reshape(1, _D)

    degp = _deg_call(dst3)
    p_spec = pl.BlockSpec((_NC, _ROWS), lambda i: (0, i))

    xw1p = pl.pallas_call(
        _xw_body,
        grid=(_GRID,),
        in_specs=[_row_spec(_D), _const_spec((_D, _D)), p_spec],
        out_specs=_row_spec(_D),
        out_shape=jax.ShapeDtypeStruct((_NP, _D), jnp.float32),
    )(x, W1, degp)

    s = _make_spmm()(xw1p, packed)

    xw2p = pl.pallas_call(
        _mid_body,
        grid=(_GRID,),
        in_specs=[_s_spec, _row_spec(_D), p_spec,
                  _const_spec((1, _D)), _const_spec((_D, _D))],
        out_specs=_row_spec(_D),
        out_shape=jax.ShapeDtypeStruct((_NP, _D), jnp.float32),
    )(s, xw1p, degp, b1r, W2)

    t = _make_spmm()(xw2p, packed)

    g = pl.pallas_call(
        _pool_body,
        grid=(_GRID,),
        in_specs=[_s_spec, _row_spec(_D), p_spec,
                  _const_spec((1, _D)), _row_spec(1)],
        out_specs=_const_spec((_B, _D)),
        out_shape=jax.ShapeDtypeStruct((_B, _D), jnp.float32),
        scratch_shapes=[pltpu.VMEM((_B, _D), jnp.float32),
                        pltpu.VMEM((_B, 1), jnp.float32)],
    )(t, xw2p, degp, b2r, bid)

    return g


# speedup vs baseline: 1.0965x; 1.0283x over previous
import functools

import jax
import jax.numpy as jnp
from jax import lax
from jax.experimental import pallas as pl
from jax.experimental.pallas import tpu as pltpu
from jax.experimental.pallas import tpu_sc as plsc

_N = 10000
_E = 320000
_D = 128
_B = 16

_NC = 2
_NS = 16
_NW = _NC * _NS

_NP = 10240
_RPT = _NP // _NS

_K = 64
_NCH = 160
_EP = _NW * _NCH * _K

_EROWS = _E // 128
_EPROWS = _EP // 128
_PBLK = 320
_TROWS = _EPROWS // _NW

_ROWS = 1280
_GRID = _NP // _ROWS

_mesh = plsc.VectorSubcoreMesh(core_axis_name="c", subcore_axis_name="s")


def _prep_body(src_ref, dst_ref, pk_ref, d_ref):
    i = pl.program_id(0)
    rid = (lax.broadcasted_iota(jnp.int32, (_PBLK, 128), 0) * 128
           + lax.broadcasted_iota(jnp.int32, (_PBLK, 128), 1)
           + i * _PBLK * 128)
    valid = rid < _E
    pad_i = _N + rid % (_NP - _N)
    s = jnp.where(valid, src_ref[0], pad_i)
    d = jnp.where(valid, dst_ref[0], pad_i)
    pk_ref[...] = s | (d << 16)
    d_ref[...] = d


def _prep_call(edge3):
    return pl.pallas_call(
        _prep_body,
        grid=(_EPROWS // _PBLK,),
        in_specs=[pl.BlockSpec((1, _PBLK, 128), lambda i: (0, i, 0)),
                  pl.BlockSpec((1, _PBLK, 128), lambda i: (1, i, 0))],
        out_specs=[pl.BlockSpec((_PBLK, 128), lambda i: (i, 0)),
                   pl.BlockSpec((_PBLK, 128), lambda i: (i, 0))],
        out_shape=[jax.ShapeDtypeStruct((_EPROWS, 128), jnp.int32),
                   jax.ShapeDtypeStruct((_EPROWS, 128), jnp.int32)],
    )(edge3, edge3)


def _deg_body(dst_hbm, out_hbm, dst_v, ones_v, zero_v, acc_sh, sem):
    cid = lax.axis_index("c")
    sid = lax.axis_index("s")
    w = cid * _NS + sid
    pltpu.sync_copy(dst_hbm.at[pl.ds(w * _TROWS, _TROWS)], dst_v)
    for i in range(128 // 16):
        ones_v[pl.ds(i * 16, 16)] = jnp.ones((16,), jnp.float32)
    for i in range(_RPT // 16):
        zero_v[pl.ds(i * 16, 16)] = jnp.zeros((16,), jnp.float32)
    pltpu.sync_copy(zero_v, acc_sh.at[pl.ds(sid * _RPT, _RPT)])
    plsc.subcore_barrier()

    def fire(j, carry):
        pltpu.async_copy(ones_v, acc_sh.at[dst_v.at[j]], sem, add=True)
        return carry

    lax.fori_loop(0, _TROWS, fire, 0)

    def drain(j, carry):
        pltpu.make_async_copy(ones_v, acc_sh.at[dst_v.at[0]], sem).wait()
        return carry

    lax.fori_loop(0, _TROWS, drain, 0)
    plsc.subcore_barrier()
    pltpu.sync_copy(acc_sh.at[pl.ds(sid * _RPT, _RPT)],
                    out_hbm.at[cid, pl.ds(sid * _RPT, _RPT)])


_deg_call = functools.partial(
    pl.kernel,
    out_type=jax.ShapeDtypeStruct((_NC, _NP), jnp.float32),
    mesh=_mesh,
    scratch_types=[
        pltpu.VMEM((_TROWS, 128), jnp.int32),
        pltpu.VMEM((128,), jnp.float32),
        pltpu.VMEM((_RPT,), jnp.float32),
        pltpu.VMEM_SHARED((_NP,), jnp.float32),
        pltpu.SemaphoreType.DMA,
    ],
)(_deg_body)


def _unpack16(packed_v, j, out_ref, hi):
    r = lax.shift_right_logical(j, 1)
    c0 = lax.bitwise_and(j, 1) * _K
    for i in range(_K // 16):
        wv = packed_v[r, pl.ds(c0 + i * 16, 16)]
        if hi:
            v = lax.shift_right_logical(wv, 16)
        else:
            v = lax.bitwise_and(wv, 0xFFFF)
        out_ref[pl.ds(i * 16, 16)] = v


def _spmm_body(xw_hbm, packed_hbm, out_hbm, packed_v, src_a, src_b, src_c,
               dst_c, buf_a, buf_b, buf_c, acc_sh, sem_a, sem_b, sem_c):
    cid = lax.axis_index("c")
    sid = lax.axis_index("s")
    w = cid * _NS + sid
    pltpu.sync_copy(packed_hbm.at[pl.ds(w * _TROWS, _TROWS)], packed_v)

    bufs = (buf_a, buf_b, buf_c)
    srcs = (src_a, src_b, src_c)
    sems = (sem_a, sem_b, sem_c)

    def gstart(j, b):
        _unpack16(packed_v, j, srcs[b], hi=False)
        pltpu.async_copy(xw_hbm.at[srcs[b]], bufs[b], sems[b])

    def gwait(b):
        pltpu.make_async_copy(xw_hbm.at[srcs[b]], bufs[b], sems[b]).wait()

    gstart(0, 0)
    gstart(1, 1)
    gstart(2, 2)
    pltpu.sync_copy(xw_hbm.at[pl.ds(sid * _RPT, _RPT)],
                    acc_sh.at[pl.ds(sid * _RPT, _RPT)])
    plsc.subcore_barrier()

    def outer(g, carry):
        j0 = 3 * g
        for b in range(3):
            j = j0 + b
            gwait(b)
            _unpack16(packed_v, j, dst_c, hi=True)
            pltpu.sync_copy(bufs[b], acc_sh.at[dst_c], add=True)
            nj = j + 3

            @pl.when(nj < _NCH)
            def _():
                gstart(nj, b)
        return carry

    lax.fori_loop(0, _NCH // 3, outer, 0)
    gwait(0)
    _unpack16(packed_v, _NCH - 1, dst_c, hi=True)
    pltpu.sync_copy(bufs[0], acc_sh.at[dst_c], add=True)
    plsc.subcore_barrier()
    pltpu.sync_copy(acc_sh.at[pl.ds(sid * _RPT, _RPT)],
                    out_hbm.at[cid, pl.ds(sid * _RPT, _RPT)])


def _make_spmm():
    return functools.partial(
        pl.kernel,
        out_type=jax.ShapeDtypeStruct((_NC, _NP, _D), jnp.float32),
        mesh=_mesh,
        scratch_types=[
            pltpu.VMEM((_TROWS, 128), jnp.int32),
            pltpu.VMEM((_K,), jnp.int32),
            pltpu.VMEM((_K,), jnp.int32),
            pltpu.VMEM((_K,), jnp.int32),
            pltpu.VMEM((_K,), jnp.int32),
            pltpu.VMEM((_K, _D), jnp.float32),
            pltpu.VMEM((_K, _D), jnp.float32),
            pltpu.VMEM((_K, _D), jnp.float32),
            pltpu.VMEM_SHARED((_NP, _D), jnp.float32),
            pltpu.SemaphoreType.DMA,
            pltpu.SemaphoreType.DMA,
            pltpu.SemaphoreType.DMA,
        ],
    )(_spmm_body)


def _dinv_col(p_ref):
    d = lax.rsqrt(1.0 + p_ref[0:1, :] + p_ref[1:2, :])
    return jnp.transpose(d, (1, 0))


def _xw_body(x_ref, w_ref, p_ref, o_ref):
    dinv = _dinv_col(p_ref)
    o_ref[...] = jnp.dot(x_ref[...], w_ref[...],
                         preferred_element_type=jnp.float32) * dinv


def _mid_body(s_ref, xwp_ref, p_ref, b_ref, w_ref, o_ref):
    dinv = _dinv_col(p_ref)
    z = (s_ref[0] + s_ref[1] - xwp_ref[...]) * dinv + b_ref[...]
    h = jnp.where(z > 0, z, jnp.exp(z) - 1.0)
    o_ref[...] = jnp.dot(h, w_ref[...],
                         preferred_element_type=jnp.float32) * dinv


def _pool_body(s_ref, xwp_ref, p_ref, b_ref, bid_ref,
               g_ref, acc, cnt):
    i = pl.program_id(0)

    @pl.when(i == 0)
    def _init():
        acc[...] = jnp.zeros_like(acc)
        cnt[...] = jnp.zeros_like(cnt)

    dinv = _dinv_col(p_ref)
    z = (s_ref[0] + s_ref[1] - xwp_ref[...]) * dinv + b_ref[...]
    h = jnp.where(z > 0, z, jnp.exp(z) - 1.0)
    rid = lax.broadcasted_iota(jnp.int32, (_ROWS, 1), 0) + i * _ROWS
    live = (rid < _N).astype(jnp.float32)
    h = h * live
    onehot = (bid_ref[...] ==
              lax.broadcasted_iota(jnp.int32, (_ROWS, _B), 1)).astype(
                  jnp.float32)
    acc[...] += lax.dot_general(onehot, h, (((0,), (0,)), ((), ())),
                                preferred_element_type=jnp.float32)
    cnt[...] += lax.dot_general(onehot, live, (((0,), (0,)), ((), ())),
                                preferred_element_type=jnp.float32)

    @pl.when(i == _GRID - 1)
    def _fin():
        g_ref[...] = acc[...] / jnp.maximum(cnt[...], 1.0)


def _row_spec(cols):
    return pl.BlockSpec((_ROWS, cols), lambda i: (i, 0))


def _const_spec(shape):
    return pl.BlockSpec(shape, lambda i: (0, 0))


_s_spec = pl.BlockSpec((_NC, _ROWS, _D), lambda i: (0, i, 0))


def kernel(x, edge_index, batch, W1, b1, W2, b2):
    edge3 = edge_index.reshape(2, _EROWS, 128)
    packed, dst3 = _prep_call(edge3)
    bid = batch.reshape(_N, 1)
    b1r = b1.reshape(1, _D)
    b2r = b2.reshape(1, _D)

    degp = _deg_call(dst3)
    p_spec = pl.BlockSpec((_NC, _ROWS), lambda i: (0, i))

    xw1p = pl.pallas_call(
        _xw_body,
        grid=(_GRID,),
        in_specs=[_row_spec(_D), _const_spec((_D, _D)), p_spec],
        out_specs=_row_spec(_D),
        out_shape=jax.ShapeDtypeStruct((_NP, _D), jnp.float32),
    )(x, W1, degp)

    s = _make_spmm()(xw1p, packed)

    xw2p = pl.pallas_call(
        _mid_body,
        grid=(_GRID,),
        in_specs=[_s_spec, _row_spec(_D), p_spec,
                  _const_spec((1, _D)), _const_spec((_D, _D))],
        out_specs=_row_spec(_D),
        out_shape=jax.ShapeDtypeStruct((_NP, _D), jnp.float32),
    )(s, xw1p, degp, b1r, W2)

    t = _make_spmm()(xw2p, packed)

    g = pl.pallas_call(
        _pool_body,
        grid=(_GRID,),
        in_specs=[_s_spec, _row_spec(_D), p_spec,
                  _const_spec((1, _D)), _row_spec(1)],
        out_specs=_const_spec((_B, _D)),
        out_shape=jax.ShapeDtypeStruct((_B, _D), jnp.float32),
        scratch_shapes=[pltpu.VMEM((_B, _D), jnp.float32),
                        pltpu.VMEM((_B, 1), jnp.float32)],
    )(t, xw2p, degp, b2r, bid)

    return g


# speedup vs baseline: 1.1234x; 1.0246x over previous
import functools

import jax
import jax.numpy as jnp
from jax import lax
from jax.experimental import pallas as pl
from jax.experimental.pallas import tpu as pltpu
from jax.experimental.pallas import tpu_sc as plsc

_N = 10000
_E = 320000
_D = 128
_B = 16

_NC = 2
_NS = 16
_NW = _NC * _NS

_NP = 10240
_RPT = _NP // _NS

_K = 64
_NCH = 160
_EP = _NW * _NCH * _K

_EROWS = _E // 128
_EPROWS = _EP // 128
_PBLK = 320
_TROWS = _EPROWS // _NW

_ROWS = 2560
_GRID = _NP // _ROWS

_mesh = plsc.VectorSubcoreMesh(core_axis_name="c", subcore_axis_name="s")


def _prep_body(src_ref, dst_ref, pk_ref, d_ref):
    i = pl.program_id(0)
    rid = (lax.broadcasted_iota(jnp.int32, (_PBLK, 128), 0) * 128
           + lax.broadcasted_iota(jnp.int32, (_PBLK, 128), 1)
           + i * _PBLK * 128)
    valid = rid < _E
    pad_i = _N + rid % (_NP - _N)
    s = jnp.where(valid, src_ref[0], pad_i)
    d = jnp.where(valid, dst_ref[0], pad_i)
    pk_ref[...] = s | (d << 16)
    d_ref[...] = d


def _prep_call(edge3):
    return pl.pallas_call(
        _prep_body,
        grid=(_EPROWS // _PBLK,),
        in_specs=[pl.BlockSpec((1, _PBLK, 128), lambda i: (0, i, 0)),
                  pl.BlockSpec((1, _PBLK, 128), lambda i: (1, i, 0))],
        out_specs=[pl.BlockSpec((_PBLK, 128), lambda i: (i, 0)),
                   pl.BlockSpec((_PBLK, 128), lambda i: (i, 0))],
        out_shape=[jax.ShapeDtypeStruct((_EPROWS, 128), jnp.int32),
                   jax.ShapeDtypeStruct((_EPROWS, 128), jnp.int32)],
    )(edge3, edge3)


def _deg_body(dst_hbm, out_hbm, dst_v, ones_v, zero_v, acc_sh, sem):
    cid = lax.axis_index("c")
    sid = lax.axis_index("s")
    w = cid * _NS + sid
    pltpu.sync_copy(dst_hbm.at[pl.ds(w * _TROWS, _TROWS)], dst_v)
    for i in range(128 // 16):
        ones_v[pl.ds(i * 16, 16)] = jnp.ones((16,), jnp.float32)
    for i in range(_RPT // 16):
        zero_v[pl.ds(i * 16, 16)] = jnp.zeros((16,), jnp.float32)
    pltpu.sync_copy(zero_v, acc_sh.at[pl.ds(sid * _RPT, _RPT)])
    plsc.subcore_barrier()

    def fire(j, carry):
        pltpu.async_copy(ones_v, acc_sh.at[dst_v.at[j]], sem, add=True)
        return carry

    lax.fori_loop(0, _TROWS, fire, 0)

    def drain(j, carry):
        pltpu.make_async_copy(ones_v, acc_sh.at[dst_v.at[0]], sem).wait()
        return carry

    lax.fori_loop(0, _TROWS, drain, 0)
    plsc.subcore_barrier()
    pltpu.sync_copy(acc_sh.at[pl.ds(sid * _RPT, _RPT)],
                    out_hbm.at[cid, pl.ds(sid * _RPT, _RPT)])


_deg_call = functools.partial(
    pl.kernel,
    out_type=jax.ShapeDtypeStruct((_NC, _NP), jnp.float32),
    mesh=_mesh,
    scratch_types=[
        pltpu.VMEM((_TROWS, 128), jnp.int32),
        pltpu.VMEM((128,), jnp.float32),
        pltpu.VMEM((_RPT,), jnp.float32),
        pltpu.VMEM_SHARED((_NP,), jnp.float32),
        pltpu.SemaphoreType.DMA,
    ],
)(_deg_body)


def _unpack16(packed_v, j, out_ref, hi):
    r = lax.shift_right_logical(j, 1)
    c0 = lax.bitwise_and(j, 1) * _K
    for i in range(_K // 16):
        wv = packed_v[r, pl.ds(c0 + i * 16, 16)]
        if hi:
            v = lax.shift_right_logical(wv, 16)
        else:
            v = lax.bitwise_and(wv, 0xFFFF)
        out_ref[pl.ds(i * 16, 16)] = v


def _spmm_body(xw_hbm, packed_hbm, out_hbm, packed_v, src_a, src_b, src_c,
               dst_c, buf_a, buf_b, buf_c, acc_sh, sem_a, sem_b, sem_c):
    cid = lax.axis_index("c")
    sid = lax.axis_index("s")
    w = cid * _NS + sid
    pltpu.sync_copy(packed_hbm.at[pl.ds(w * _TROWS, _TROWS)], packed_v)

    bufs = (buf_a, buf_b, buf_c)
    srcs = (src_a, src_b, src_c)
    sems = (sem_a, sem_b, sem_c)

    def gstart(j, b):
        _unpack16(packed_v, j, srcs[b], hi=False)
        pltpu.async_copy(xw_hbm.at[srcs[b]], bufs[b], sems[b])

    def gwait(b):
        pltpu.make_async_copy(xw_hbm.at[srcs[b]], bufs[b], sems[b]).wait()

    gstart(0, 0)
    gstart(1, 1)
    gstart(2, 2)
    pltpu.sync_copy(xw_hbm.at[pl.ds(sid * _RPT, _RPT)],
                    acc_sh.at[pl.ds(sid * _RPT, _RPT)])
    plsc.subcore_barrier()

    def outer(g, carry):
        j0 = 3 * g
        for b in range(3):
            j = j0 + b
            gwait(b)
            _unpack16(packed_v, j, dst_c, hi=True)
            pltpu.sync_copy(bufs[b], acc_sh.at[dst_c], add=True)
            nj = j + 3

            @pl.when(nj < _NCH)
            def _():
                gstart(nj, b)
        return carry

    lax.fori_loop(0, _NCH // 3, outer, 0)
    gwait(0)
    _unpack16(packed_v, _NCH - 1, dst_c, hi=True)
    pltpu.sync_copy(bufs[0], acc_sh.at[dst_c], add=True)
    plsc.subcore_barrier()
    pltpu.sync_copy(acc_sh.at[pl.ds(sid * _RPT, _RPT)],
                    out_hbm.at[cid, pl.ds(sid * _RPT, _RPT)])


def _make_spmm():
    return functools.partial(
        pl.kernel,
        out_type=jax.ShapeDtypeStruct((_NC, _NP, _D), jnp.float32),
        mesh=_mesh,
        scratch_types=[
            pltpu.VMEM((_TROWS, 128), jnp.int32),
            pltpu.VMEM((_K,), jnp.int32),
            pltpu.VMEM((_K,), jnp.int32),
            pltpu.VMEM((_K,), jnp.int32),
            pltpu.VMEM((_K,), jnp.int32),
            pltpu.VMEM((_K, _D), jnp.float32),
            pltpu.VMEM((_K, _D), jnp.float32),
            pltpu.VMEM((_K, _D), jnp.float32),
            pltpu.VMEM_SHARED((_NP, _D), jnp.float32),
            pltpu.SemaphoreType.DMA,
            pltpu.SemaphoreType.DMA,
            pltpu.SemaphoreType.DMA,
        ],
    )(_spmm_body)


def _dinv_col(p_ref):
    d = lax.rsqrt(1.0 + p_ref[0:1, :] + p_ref[1:2, :])
    return jnp.transpose(d, (1, 0))


def _xw_body(x_ref, w_ref, p_ref, o_ref):
    dinv = _dinv_col(p_ref)
    o_ref[...] = jnp.dot(x_ref[...], w_ref[...],
                         preferred_element_type=jnp.float32) * dinv


def _mid_body(s_ref, xwp_ref, p_ref, b_ref, w_ref, o_ref):
    dinv = _dinv_col(p_ref)
    z = (s_ref[0] + s_ref[1] - xwp_ref[...]) * dinv + b_ref[...]
    h = jnp.where(z > 0, z, jnp.exp(z) - 1.0)
    o_ref[...] = jnp.dot(h, w_ref[...],
                         preferred_element_type=jnp.float32) * dinv


def _pool_body(s_ref, xwp_ref, p_ref, b_ref, bid_ref,
               g_ref, acc, cnt):
    i = pl.program_id(0)

    @pl.when(i == 0)
    def _init():
        acc[...] = jnp.zeros_like(acc)
        cnt[...] = jnp.zeros_like(cnt)

    dinv = _dinv_col(p_ref)
    z = (s_ref[0] + s_ref[1] - xwp_ref[...]) * dinv + b_ref[...]
    h = jnp.where(z > 0, z, jnp.exp(z) - 1.0)
    rid = lax.broadcasted_iota(jnp.int32, (_ROWS, 1), 0) + i * _ROWS
    live = (rid < _N).astype(jnp.float32)
    h = h * live
    onehot = (bid_ref[...] ==
              lax.broadcasted_iota(jnp.int32, (_ROWS, _B), 1)).astype(
                  jnp.float32)
    acc[...] += lax.dot_general(onehot, h, (((0,), (0,)), ((), ())),
                                preferred_element_type=jnp.float32)
    cnt[...] += lax.dot_general(onehot, live, (((0,), (0,)), ((), ())),
                                preferred_element_type=jnp.float32)

    @pl.when(i == _GRID - 1)
    def _fin():
        g_ref[...] = acc[...] / jnp.maximum(cnt[...], 1.0)


def _row_spec(cols):
    return pl.BlockSpec((_ROWS, cols), lambda i: (i, 0))


def _const_spec(shape):
    return pl.BlockSpec(shape, lambda i: (0, 0))


_s_spec = pl.BlockSpec((_NC, _ROWS, _D), lambda i: (0, i, 0))


def kernel(x, edge_index, batch, W1, b1, W2, b2):
    edge3 = edge_index.reshape(2, _EROWS, 128)
    packed, dst3 = _prep_call(edge3)
    bid = batch.reshape(_N, 1)
    b1r = b1.reshape(1, _D)
    b2r = b2.reshape(1, _D)

    degp = _deg_call(dst3)
    p_spec = pl.BlockSpec((_NC, _ROWS), lambda i: (0, i))

    xw1p = pl.pallas_call(
        _xw_body,
        grid=(_GRID,),
        in_specs=[_row_spec(_D), _const_spec((_D, _D)), p_spec],
        out_specs=_row_spec(_D),
        out_shape=jax.ShapeDtypeStruct((_NP, _D), jnp.float32),
    )(x, W1, degp)

    s = _make_spmm()(xw1p, packed)

    xw2p = pl.pallas_call(
        _mid_body,
        grid=(_GRID,),
        in_specs=[_s_spec, _row_spec(_D), p_spec,
                  _const_spec((1, _D)), _const_spec((_D, _D))],
        out_specs=_row_spec(_D),
        out_shape=jax.ShapeDtypeStruct((_NP, _D), jnp.float32),
    )(s, xw1p, degp, b1r, W2)

    t = _make_spmm()(xw2p, packed)

    g = pl.pallas_call(
        _pool_body,
        grid=(_GRID,),
        in_specs=[_s_spec, _row_spec(_D), p_spec,
                  _const_spec((1, _D)), _row_spec(1)],
        out_specs=_const_spec((_B, _D)),
        out_shape=jax.ShapeDtypeStruct((_B, _D), jnp.float32),
        scratch_shapes=[pltpu.VMEM((_B, _D), jnp.float32),
                        pltpu.VMEM((_B, 1), jnp.float32)],
    )(t, xw2p, degp, b2r, bid)

    return g
